# baseline (speedup 1.0000x reference)
import numpy as np
import jax
import jax.numpy as jnp
from jax.experimental import pallas as pl
from jax.experimental.pallas import tpu as pltpu

_H = 900
_W = 900
_B = 128
_N = _B * _B


def _init_graph_np(valid, step=1):
    h, w = valid.shape
    u = np.arange(0, w - 1, step)
    v = np.arange(0, h - 1, step)
    u, v = np.meshgrid(u, v, indexing="xy")
    anchor_valid = valid[v, u]
    u = u[anchor_valid]
    v = v[anchor_valid]
    index_map = -np.ones((h, w), dtype=np.int64)
    index_map[v, u] = np.arange(len(u))

    base = np.stack([u, v], axis=1).reshape(-1, 1, 1, 2)
    edges = np.tile(base, (1, 4, 2, 1)).copy()
    edges[:, 0, 1, 0] += step
    edges[:, 1, 1, 0] += step
    edges[:, 1, 1, 1] += step
    edges[:, 2, 1, 1] += step
    edges[:, 3, 0, 0] += step
    edges[:, 3, 1, 1] += step

    faces = np.concatenate(
        [np.tile(base, (1, 2, 1, 1)),
         np.stack([edges[:, 0:2, 1, :], edges[:, 1:3, 1, :]], axis=1)],
        axis=2)

    valid_p = np.pad(valid, ((0, step), (0, step)), constant_values=False)

    edge_ok = ~np.any(~valid_p[edges[..., 1], edges[..., 0]], axis=2)
    edges = edges[edge_ok]
    edges = index_map[edges[..., 1], edges[..., 0]]
    edges = edges[~np.any(edges < 0, axis=1)]

    face_ok = ~np.any(~valid_p[faces[..., 1], faces[..., 0]], axis=2)
    faces = faces[face_ok]
    faces = index_map[faces[..., 1], faces[..., 0]]
    faces = faces[~np.any(faces < 0, axis=1)]

    return (index_map >= 0), edges.T, faces.T


_valid_np = np.zeros((_H, _W), dtype=bool)
_valid_np[:_B, :_B] = True
_val_bool_np, _edge_index_np, _triangles_np = _init_graph_np(_valid_np, step=1)
_E = int(_edge_index_np.shape[1])
_T = int(_triangles_np.shape[1])

_deg_np = np.bincount(_edge_index_np.reshape(-1).astype(np.int64),
                      minlength=_N)[:_N].astype(np.float32)
assert np.all(_deg_np > 0.0)
_DEG_GRID = _deg_np.reshape(_B, _B)

_n0, _n1 = _edge_index_np[0].astype(np.int64), _edge_index_np[1].astype(np.int64)
_v0, _u0 = _n0 // _B, _n0 % _B
_v1, _u1 = _n1 // _B, _n1 % _B
_du, _dv = _u1 - _u0, _v1 - _v0
_etype = np.full(_E, -1, dtype=np.int64)
_etype[(_du == 1) & (_dv == 0)] = 0
_etype[(_du == 1) & (_dv == 1)] = 1
_etype[(_du == 0) & (_dv == 1)] = 2
_etype[(_du == -1) & (_dv == 1)] = 3
assert np.all(_etype >= 0)
_au = np.where(_etype == 3, _u0 - 1, _u0)
_av = _v0
_PERM_E_NP = (_etype * _N + _av * _B + _au).astype(np.int32)
assert _PERM_E_NP.min() >= 0 and _PERM_E_NP.max() < 4 * _N

_f0, _f1 = _triangles_np[0].astype(np.int64), _triangles_np[1].astype(np.int64)
_which = np.full(_T, -1, dtype=np.int64)
_which[(_f1 - _f0) == 1] = 0
_which[(_f1 - _f0) == _B + 1] = 1
assert np.all(_which >= 0)
_PERM_T_NP = (_which * _N + _f0).astype(np.int32)
assert _PERM_T_NP.min() >= 0 and _PERM_T_NP.max() < 2 * _N


def _grid_kernel(p_ref, deg_ref, norms_ref, lens_ref, areas_ref, radii_ref,
                 ednorms_ref, sem):
    copies = [pltpu.make_async_copy(norms_ref.at[pl.ds(v * _W, _B), :],
                                    ednorms_ref.at[pl.ds(v * _B, _B), :],
                                    sem)
              for v in range(_B)]
    for c in copies:
        c.start()

    x = p_ref[0]
    y = p_ref[1]
    z = p_ref[2]

    def shl(a):
        return pltpu.roll(a, _B - 1, 1)

    def shu(a):
        return pltpu.roll(a, _B - 1, 0)

    def shr(a):
        return pltpu.roll(a, 1, 1)

    def shd(a):
        return pltpu.roll(a, 1, 0)

    xr, yr, zr = shl(x), shl(y), shl(z)
    xd, yd, zd = shu(x), shu(y), shu(z)
    xq, yq, zq = shl(xd), shl(yd), shl(zd)

    uu = jax.lax.broadcasted_iota(jnp.int32, (_B, _B), 1)
    vv = jax.lax.broadcasted_iota(jnp.int32, (_B, _B), 0)
    mu = uu < (_B - 1)
    mv = vv < (_B - 1)
    muv = mu & mv

    def dist(ax, ay, az, bx, by, bz):
        dx = ax - bx
        dy = ay - by
        dz = az - bz
        return jnp.sqrt(dx * dx + dy * dy + dz * dz)

    zero = jnp.float32(0.0)
    l0 = jnp.where(mu, dist(x, y, z, xr, yr, zr), zero)
    l1 = jnp.where(muv, dist(x, y, z, xq, yq, zq), zero)
    l2 = jnp.where(mv, dist(x, y, z, xd, yd, zd), zero)
    l3 = jnp.where(muv, dist(xr, yr, zr, xd, yd, zd), zero)
    lens_ref[0] = l0
    lens_ref[1] = l1
    lens_ref[2] = l2
    lens_ref[3] = l3

    sums = ((l0 + shr(l0)) + (l1 + shd(shr(l1)))
            + (l2 + shd(l2)) + (shr(l3) + shd(l3)))
    radii_ref[...] = sums / deg_ref[...]

    def area(ax, ay, az, bx, by, bz):
        cx = ay * bz - az * by
        cy = az * bx - ax * bz
        cz = ax * by - ay * bx
        return 0.5 * jnp.sqrt(cx * cx + cy * cy + cz * cz + 1e-13)

    a0 = area(xr - x, yr - y, zr - z, xq - x, yq - y, zq - z)
    a1 = area(xq - x, yq - y, zq - z, xd - x, yd - y, zd - z)
    areas_ref[0] = jnp.where(muv, a0, zero)
    areas_ref[1] = jnp.where(muv, a1, zero)

    for c in copies:
        c.wait()


def _run_grid_kernel(p3, deg, norms):
    return pl.pallas_call(
        _grid_kernel,
        out_shape=(jax.ShapeDtypeStruct((4, _B, _B), jnp.float32),
                   jax.ShapeDtypeStruct((2, _B, _B), jnp.float32),
                   jax.ShapeDtypeStruct((_B, _B), jnp.float32),
                   jax.ShapeDtypeStruct((_N, 3), jnp.float32)),
        in_specs=[pl.BlockSpec((3, _B, _B), lambda: (0, 0, 0)),
                  pl.BlockSpec((_B, _B), lambda: (0, 0)),
                  pl.BlockSpec(memory_space=pl.ANY)],
        out_specs=[pl.BlockSpec((4, _B, _B), lambda: (0, 0, 0)),
                   pl.BlockSpec((2, _B, _B), lambda: (0, 0, 0)),
                   pl.BlockSpec((_B, _B), lambda: (0, 0)),
                   pl.BlockSpec(memory_space=pl.ANY)],
        scratch_shapes=[pltpu.SemaphoreType.DMA],
        compiler_params=pltpu.CompilerParams(vmem_limit_bytes=32 << 20),
    )(p3, deg, norms)


_VAL_IDX_NP = (np.arange(_B)[:, None] * _W + np.arange(_B)[None, :]).reshape(-1)
_VAL_IDX_NP = _VAL_IDX_NP.astype(np.int32)


def kernel(points, norms):
    val_idx = jnp.asarray(_VAL_IDX_NP)
    ed_points = jnp.take(points, val_idx, axis=0).astype(jnp.float32)

    p3 = jnp.transpose(ed_points, (1, 0)).reshape(3, _B, _B)
    deg = jnp.asarray(_DEG_GRID)

    lens, areas, radii_grid, ed_norms = _run_grid_kernel(p3, deg, norms)

    full = jnp.transpose(lens, (1, 2, 0)).reshape(_B, 4 * _B)
    rows = jnp.concatenate(
        [full[:_B - 1, :4 * (_B - 1)],
         full[:_B - 1, 4 * (_B - 1) + 2:4 * (_B - 1) + 3]],
        axis=1)
    tail = jax.lax.slice(full, (_B - 1, 0), (_B, 4 * (_B - 1)), (1, 4))
    edges_lens = jnp.concatenate([rows.reshape(-1), tail.reshape(-1)], axis=0)

    tri_areas = (jnp.transpose(areas, (1, 2, 0)).reshape(_B, 2 * _B)
                 [:_B - 1, :2 * (_B - 1)].reshape(-1))
    radii = radii_grid.reshape(_N)

    return {
        "points": ed_points,
        "norms": ed_norms,
        "radii": radii,
        "edge_index": jnp.asarray(_edge_index_np.astype(np.int32)),
        "edges_lens": edges_lens,
        "triangles": jnp.asarray(_triangles_np.astype(np.int32)),
        "triangles_areas": tri_areas,
        "num": _N,
        "param_num": _N * 7,
    }


# speedup vs baseline: 5.3780x; 5.3780x over previous
import numpy as np
import jax
import jax.numpy as jnp
from jax.experimental import pallas as pl
from jax.experimental.pallas import tpu as pltpu

_H = 900
_W = 900
_B = 128
_N = _B * _B


def _init_graph_np(valid, step=1):
    h, w = valid.shape
    u = np.arange(0, w - 1, step)
    v = np.arange(0, h - 1, step)
    u, v = np.meshgrid(u, v, indexing="xy")
    anchor_valid = valid[v, u]
    u = u[anchor_valid]
    v = v[anchor_valid]
    index_map = -np.ones((h, w), dtype=np.int64)
    index_map[v, u] = np.arange(len(u))

    base = np.stack([u, v], axis=1).reshape(-1, 1, 1, 2)
    edges = np.tile(base, (1, 4, 2, 1)).copy()
    edges[:, 0, 1, 0] += step
    edges[:, 1, 1, 0] += step
    edges[:, 1, 1, 1] += step
    edges[:, 2, 1, 1] += step
    edges[:, 3, 0, 0] += step
    edges[:, 3, 1, 1] += step

    faces = np.concatenate(
        [np.tile(base, (1, 2, 1, 1)),
         np.stack([edges[:, 0:2, 1, :], edges[:, 1:3, 1, :]], axis=1)],
        axis=2)

    valid_p = np.pad(valid, ((0, step), (0, step)), constant_values=False)

    edge_ok = ~np.any(~valid_p[edges[..., 1], edges[..., 0]], axis=2)
    edges = edges[edge_ok]
    edges = index_map[edges[..., 1], edges[..., 0]]
    edges = edges[~np.any(edges < 0, axis=1)]

    face_ok = ~np.any(~valid_p[faces[..., 1], faces[..., 0]], axis=2)
    faces = faces[face_ok]
    faces = index_map[faces[..., 1], faces[..., 0]]
    faces = faces[~np.any(faces < 0, axis=1)]

    return (index_map >= 0), edges.T, faces.T


_valid_np = np.zeros((_H, _W), dtype=bool)
_valid_np[:_B, :_B] = True
_val_bool_np, _edge_index_np, _triangles_np = _init_graph_np(_valid_np, step=1)
_E = int(_edge_index_np.shape[1])
_T = int(_triangles_np.shape[1])

_deg_np = np.bincount(_edge_index_np.reshape(-1).astype(np.int64),
                      minlength=_N)[:_N].astype(np.float32)
assert np.all(_deg_np > 0.0)
_DEG_GRID = _deg_np.reshape(_B, _B)

_n0, _n1 = _edge_index_np[0].astype(np.int64), _edge_index_np[1].astype(np.int64)
_v0, _u0 = _n0 // _B, _n0 % _B
_v1, _u1 = _n1 // _B, _n1 % _B
_du, _dv = _u1 - _u0, _v1 - _v0
_etype = np.full(_E, -1, dtype=np.int64)
_etype[(_du == 1) & (_dv == 0)] = 0
_etype[(_du == 1) & (_dv == 1)] = 1
_etype[(_du == 0) & (_dv == 1)] = 2
_etype[(_du == -1) & (_dv == 1)] = 3
assert np.all(_etype >= 0)
_au = np.where(_etype == 3, _u0 - 1, _u0)
_av = _v0
_PERM_E_NP = (_etype * _N + _av * _B + _au).astype(np.int32)
assert _PERM_E_NP.min() >= 0 and _PERM_E_NP.max() < 4 * _N

_f0, _f1 = _triangles_np[0].astype(np.int64), _triangles_np[1].astype(np.int64)
_which = np.full(_T, -1, dtype=np.int64)
_which[(_f1 - _f0) == 1] = 0
_which[(_f1 - _f0) == _B + 1] = 1
assert np.all(_which >= 0)
_PERM_T_NP = (_which * _N + _f0).astype(np.int32)
assert _PERM_T_NP.min() >= 0 and _PERM_T_NP.max() < 2 * _N


def _grid_kernel(p_ref, deg_ref, lens_ref, areas_ref, radii_ref):
    x = p_ref[0]
    y = p_ref[1]
    z = p_ref[2]

    def shl(a):
        return pltpu.roll(a, _B - 1, 1)

    def shu(a):
        return pltpu.roll(a, _B - 1, 0)

    def shr(a):
        return pltpu.roll(a, 1, 1)

    def shd(a):
        return pltpu.roll(a, 1, 0)

    xr, yr, zr = shl(x), shl(y), shl(z)
    xd, yd, zd = shu(x), shu(y), shu(z)
    xq, yq, zq = shl(xd), shl(yd), shl(zd)

    uu = jax.lax.broadcasted_iota(jnp.int32, (_B, _B), 1)
    vv = jax.lax.broadcasted_iota(jnp.int32, (_B, _B), 0)
    mu = uu < (_B - 1)
    mv = vv < (_B - 1)
    muv = mu & mv

    def dist(ax, ay, az, bx, by, bz):
        dx = ax - bx
        dy = ay - by
        dz = az - bz
        return jnp.sqrt(dx * dx + dy * dy + dz * dz)

    zero = jnp.float32(0.0)
    l0 = jnp.where(mu, dist(x, y, z, xr, yr, zr), zero)
    l1 = jnp.where(muv, dist(x, y, z, xq, yq, zq), zero)
    l2 = jnp.where(mv, dist(x, y, z, xd, yd, zd), zero)
    l3 = jnp.where(muv, dist(xr, yr, zr, xd, yd, zd), zero)
    lens_ref[0] = jnp.where(mu, l0, l2)
    lens_ref[1] = l1
    lens_ref[2] = l2
    lens_ref[3] = l3

    sums = ((l0 + shr(l0)) + (l1 + shd(shr(l1)))
            + (l2 + shd(l2)) + (shr(l3) + shd(l3)))
    radii_ref[...] = sums / deg_ref[...]

    def area(ax, ay, az, bx, by, bz):
        cx = ay * bz - az * by
        cy = az * bx - ax * bz
        cz = ax * by - ay * bx
        return 0.5 * jnp.sqrt(cx * cx + cy * cy + cz * cz + 1e-13)

    a0 = area(xr - x, yr - y, zr - z, xq - x, yq - y, zq - z)
    a1 = area(xq - x, yq - y, zq - z, xd - x, yd - y, zd - z)
    areas_ref[0] = jnp.where(muv, a0, zero)
    areas_ref[1] = jnp.where(muv, a1, zero)


def _run_grid_kernel(p3, deg):
    return pl.pallas_call(
        _grid_kernel,
        out_shape=(jax.ShapeDtypeStruct((4, _B, _B), jnp.float32),
                   jax.ShapeDtypeStruct((2, _B, _B), jnp.float32),
                   jax.ShapeDtypeStruct((_B, _B), jnp.float32)),
        compiler_params=pltpu.CompilerParams(vmem_limit_bytes=32 << 20),
    )(p3, deg)


_VAL_IDX_NP = (np.arange(_B)[:, None] * _W + np.arange(_B)[None, :]).reshape(-1)
_VAL_IDX_NP = _VAL_IDX_NP.astype(np.int32)


def kernel(points, norms):
    val_idx = jnp.asarray(_VAL_IDX_NP)
    ed_points = jnp.take(points, val_idx, axis=0).astype(jnp.float32)
    ed_norms = jnp.take(norms, val_idx, axis=0)

    p3 = jnp.transpose(ed_points, (1, 0)).reshape(3, _B, _B)
    deg = jnp.asarray(_DEG_GRID)

    lens, areas, radii_grid = _run_grid_kernel(p3, deg)

    full = jnp.transpose(lens, (1, 2, 0)).reshape(_B, 4 * _B)
    rows = full[:_B - 1, :4 * (_B - 1) + 1]
    tail = jax.lax.slice(full, (_B - 1, 0), (_B, 4 * (_B - 1)), (1, 4))
    edges_lens = jnp.concatenate([rows.reshape(-1), tail.reshape(-1)], axis=0)

    tri_areas = (jnp.transpose(areas, (1, 2, 0)).reshape(_B, 2 * _B)
                 [:_B - 1, :2 * (_B - 1)].reshape(-1))
    radii = radii_grid.reshape(_N)

    return {
        "points": ed_points,
        "norms": ed_norms,
        "radii": radii,
        "edge_index": jnp.asarray(_edge_index_np.astype(np.int32)),
        "edges_lens": edges_lens,
        "triangles": jnp.asarray(_triangles_np.astype(np.int32)),
        "triangles_areas": tri_areas,
        "num": _N,
        "param_num": _N * 7,
    }


# speedup vs baseline: 5.3855x; 1.0014x over previous
import numpy as np
import jax
import jax.numpy as jnp
from jax.experimental import pallas as pl
from jax.experimental.pallas import tpu as pltpu

_H = 900
_W = 900
_B = 128
_N = _B * _B


def _init_graph_np(valid, step=1):
    h, w = valid.shape
    u = np.arange(0, w - 1, step)
    v = np.arange(0, h - 1, step)
    u, v = np.meshgrid(u, v, indexing="xy")
    anchor_valid = valid[v, u]
    u = u[anchor_valid]
    v = v[anchor_valid]
    index_map = -np.ones((h, w), dtype=np.int64)
    index_map[v, u] = np.arange(len(u))

    base = np.stack([u, v], axis=1).reshape(-1, 1, 1, 2)
    edges = np.tile(base, (1, 4, 2, 1)).copy()
    edges[:, 0, 1, 0] += step
    edges[:, 1, 1, 0] += step
    edges[:, 1, 1, 1] += step
    edges[:, 2, 1, 1] += step
    edges[:, 3, 0, 0] += step
    edges[:, 3, 1, 1] += step

    faces = np.concatenate(
        [np.tile(base, (1, 2, 1, 1)),
         np.stack([edges[:, 0:2, 1, :], edges[:, 1:3, 1, :]], axis=1)],
        axis=2)

    valid_p = np.pad(valid, ((0, step), (0, step)), constant_values=False)

    edge_ok = ~np.any(~valid_p[edges[..., 1], edges[..., 0]], axis=2)
    edges = edges[edge_ok]
    edges = index_map[edges[..., 1], edges[..., 0]]
    edges = edges[~np.any(edges < 0, axis=1)]

    face_ok = ~np.any(~valid_p[faces[..., 1], faces[..., 0]], axis=2)
    faces = faces[face_ok]
    faces = index_map[faces[..., 1], faces[..., 0]]
    faces = faces[~np.any(faces < 0, axis=1)]

    return (index_map >= 0), edges.T, faces.T


_valid_np = np.zeros((_H, _W), dtype=bool)
_valid_np[:_B, :_B] = True
_val_bool_np, _edge_index_np, _triangles_np = _init_graph_np(_valid_np, step=1)
_E = int(_edge_index_np.shape[1])
_T = int(_triangles_np.shape[1])

_deg_np = np.bincount(_edge_index_np.reshape(-1).astype(np.int64),
                      minlength=_N)[:_N].astype(np.float32)
assert np.all(_deg_np > 0.0)
_DEG_GRID = _deg_np.reshape(_B, _B)

_n0, _n1 = _edge_index_np[0].astype(np.int64), _edge_index_np[1].astype(np.int64)
_v0, _u0 = _n0 // _B, _n0 % _B
_v1, _u1 = _n1 // _B, _n1 % _B
_du, _dv = _u1 - _u0, _v1 - _v0
_etype = np.full(_E, -1, dtype=np.int64)
_etype[(_du == 1) & (_dv == 0)] = 0
_etype[(_du == 1) & (_dv == 1)] = 1
_etype[(_du == 0) & (_dv == 1)] = 2
_etype[(_du == -1) & (_dv == 1)] = 3
assert np.all(_etype >= 0)
_au = np.where(_etype == 3, _u0 - 1, _u0)
_av = _v0
_pos = np.where(_av < _B - 1,
                _av * (4 * (_B - 1) + 1)
                + np.where(_au < _B - 1, 4 * _au + _etype, 4 * (_B - 1)),
                (_B - 1) * (4 * (_B - 1) + 1) + _au)
assert np.array_equal(_pos, np.arange(_E))

_f0, _f1 = _triangles_np[0].astype(np.int64), _triangles_np[1].astype(np.int64)
_which = np.full(_T, -1, dtype=np.int64)
_which[(_f1 - _f0) == 1] = 0
_which[(_f1 - _f0) == _B + 1] = 1
assert np.all(_which >= 0)
_tv, _tu = _f0 // _B, _f0 % _B
assert np.array_equal(_tv * 2 * (_B - 1) + 2 * _tu + _which, np.arange(_T))


def _grid_kernel(p_ref, deg_ref, lens_ref, areas_ref, radii_ref):
    x = p_ref[0]
    y = p_ref[1]
    z = p_ref[2]

    def shl(a):
        return pltpu.roll(a, _B - 1, 1)

    def shu(a):
        return pltpu.roll(a, _B - 1, 0)

    def shr(a):
        return pltpu.roll(a, 1, 1)

    def shd(a):
        return pltpu.roll(a, 1, 0)

    xr, yr, zr = shl(x), shl(y), shl(z)
    xd, yd, zd = shu(x), shu(y), shu(z)
    xq, yq, zq = shl(xd), shl(yd), shl(zd)

    uu = jax.lax.broadcasted_iota(jnp.int32, (_B, _B), 1)
    vv = jax.lax.broadcasted_iota(jnp.int32, (_B, _B), 0)
    mu = uu < (_B - 1)
    mv = vv < (_B - 1)
    muv = mu & mv

    def dist(ax, ay, az, bx, by, bz):
        dx = ax - bx
        dy = ay - by
        dz = az - bz
        return jnp.sqrt(dx * dx + dy * dy + dz * dz)

    zero = jnp.float32(0.0)
    l0 = jnp.where(mu, dist(x, y, z, xr, yr, zr), zero)
    l1 = jnp.where(muv, dist(x, y, z, xq, yq, zq), zero)
    l2 = jnp.where(mv, dist(x, y, z, xd, yd, zd), zero)
    l3 = jnp.where(muv, dist(xr, yr, zr, xd, yd, zd), zero)
    lens_ref[0] = jnp.where(mu, l0, l2)
    lens_ref[1] = l1
    lens_ref[2] = l2
    lens_ref[3] = l3

    sums = ((l0 + shr(l0)) + (l1 + shd(shr(l1)))
            + (l2 + shd(l2)) + (shr(l3) + shd(l3)))
    radii_ref[...] = sums / deg_ref[...]

    def area(ax, ay, az, bx, by, bz):
        cx = ay * bz - az * by
        cy = az * bx - ax * bz
        cz = ax * by - ay * bx
        return 0.5 * jnp.sqrt(cx * cx + cy * cy + cz * cz + 1e-13)

    a0 = area(xr - x, yr - y, zr - z, xq - x, yq - y, zq - z)
    a1 = area(xq - x, yq - y, zq - z, xd - x, yd - y, zd - z)
    areas_ref[0] = jnp.where(muv, a0, zero)
    areas_ref[1] = jnp.where(muv, a1, zero)


def _run_grid_kernel(p3, deg):
    return pl.pallas_call(
        _grid_kernel,
        out_shape=(jax.ShapeDtypeStruct((4, _B, _B), jnp.float32),
                   jax.ShapeDtypeStruct((2, _B, _B), jnp.float32),
                   jax.ShapeDtypeStruct((_B, _B), jnp.float32)),
        compiler_params=pltpu.CompilerParams(vmem_limit_bytes=32 << 20),
    )(p3, deg)


_VAL_IDX_NP = (np.arange(_B)[:, None] * _W + np.arange(_B)[None, :]).reshape(-1)
_VAL_IDX_NP = _VAL_IDX_NP.astype(np.int32)


def kernel(points, norms):
    val_idx = jnp.asarray(_VAL_IDX_NP)
    ed_points = jnp.take(points, val_idx, axis=0).astype(jnp.float32)
    ed_norms = jnp.take(norms, val_idx, axis=0)

    p3 = jnp.transpose(ed_points, (1, 0)).reshape(3, _B, _B)
    deg = jnp.asarray(_DEG_GRID)

    lens, areas, radii_grid = _run_grid_kernel(p3, deg)

    full = jnp.transpose(lens, (1, 2, 0)).reshape(_B, 4 * _B)
    rows = full[:_B - 1, :4 * (_B - 1) + 1]
    tail = jax.lax.slice(full, (_B - 1, 0), (_B, 4 * (_B - 1)), (1, 4))
    edges_lens = jnp.concatenate([rows.reshape(-1), tail.reshape(-1)], axis=0)

    tri_areas = (jnp.transpose(areas, (1, 2, 0)).reshape(_B, 2 * _B)
                 [:_B - 1, :2 * (_B - 1)].reshape(-1))
    radii = radii_grid.reshape(_N)

    return {
        "points": ed_points,
        "norms": ed_norms,
        "radii": radii,
        "edge_index": jnp.asarray(_edge_index_np.astype(np.int32)),
        "edges_lens": edges_lens,
        "triangles": jnp.asarray(_triangles_np.astype(np.int32)),
        "triangles_areas": tri_areas,
        "num": _N,
        "param_num": _N * 7,
    }
